# Initial kernel scaffold; baseline (speedup 1.0000x reference)
import jax
import jax.numpy as jnp
from jax.experimental import pallas as pl


def kernel(logits):
    raise NotImplementedError("write your pallas kernel here")


# speedup vs baseline: 1.1438x; 1.1438x over previous
import numpy as np
import jax
import jax.numpy as jnp
from jax import lax
from jax.experimental import pallas as pl
from jax.experimental.pallas import tpu as pltpu

N = 1_000_000
NP = 1 << 20
C = 128
R = NP // C
BR = 1024
NB = R // BR
BSZ = BR * C

_THRESH = np.float32(1.0 - 1.0 / 3.0)
_EPS = np.float32(1e-10)
_NEG = np.float32(-1e30)


def _lane_cumsum(blk):
    a = lax.broadcasted_iota(jnp.int32, (C, C), 0)
    b = lax.broadcasted_iota(jnp.int32, (C, C), 1)
    u = (a <= b).astype(jnp.float32)
    return lax.dot_general(blk, u, (((1,), (0,)), ((), ())),
                           preferred_element_type=jnp.float32)


def _row_cumsum(v):
    k = 1
    while k < BR:
        z = jnp.zeros((k, 1), v.dtype)
        v = v + jnp.concatenate([z, v[:-k]], axis=0)
        k *= 2
    return v


def _softmax_kernel(x_ref, p_ref, fs):
    s = pl.program_id(0)
    phase = s // NB

    @pl.when(s == 0)
    def _init():
        fs[0] = _NEG
        fs[1] = 0.0

    blk = x_ref[...]

    @pl.when(phase == 0)
    def _pmax():
        fs[0] = jnp.maximum(fs[0], jnp.max(blk))

    @pl.when(phase == 1)
    def _psum():
        fs[1] = fs[1] + jnp.sum(jnp.exp(blk - fs[0]))

    @pl.when(phase == 2)
    def _pwrite():
        p_ref[...] = jnp.exp(blk - fs[0]) / fs[1]


def _sample_kernel(ps_ref, g_ref, p_ref, out_ref, fs, is_):
    s = pl.program_id(0)
    phase = s // NB
    blk_id = s % NB

    @pl.when(s == 0)
    def _init():
        fs[0] = 0.0
        fs[1] = -1.0
        fs[2] = _NEG
        fs[3] = 0.0
        fs[4] = 0.0
        is_[0] = 0
        is_[1] = 0
        is_[2] = 0
        is_[3] = 0

    rr = lax.broadcasted_iota(jnp.int32, (BR, C), 0)
    cc = lax.broadcasted_iota(jnp.int32, (BR, C), 1)
    j = blk_id * BSZ + rr * C + cc

    @pl.when(phase == 0)
    def _p_cummass():
        ps = ps_ref[...]
        lc = _lane_cumsum(ps)
        rowsum = lc[:, C - 1:C]
        rp = _row_cumsum(rowsum)
        cum = fs[0] + (rp - rowsum) + lc
        kept = cum <= _THRESH
        is_[0] = is_[0] + jnp.sum(kept.astype(jnp.int32))
        fs[1] = jnp.maximum(fs[1], jnp.max(jnp.where(kept, cum, -1.0)))
        fs[0] = fs[0] + jnp.sum(rowsum)

    @pl.when(phase == 1)
    def _p_argmax():
        ps = ps_ref[...]
        g = g_ref[...]
        total = jnp.maximum(fs[1], _EPS)
        t = jnp.where(j < is_[0], ps / total, 0.0)
        sc = jnp.log(t + _EPS) + g
        bm = jnp.max(sc)
        bj = jnp.min(jnp.where(sc == bm, j, jnp.int32(2147483647)))

        @pl.when(bm > fs[2])
        def _upd():
            fs[2] = bm
            is_[1] = bj
            fs[3] = jnp.sum(jnp.where(j == bj, ps_ref[...], 0.0))

    @pl.when(phase == 2)
    def _p_gtcount():
        p = p_ref[...]
        is_[2] = is_[2] + jnp.sum((p > fs[3]).astype(jnp.int32))

    @pl.when(phase == 3)
    def _p_token():
        p = p_ref[...]
        eq = (p == fs[3]).astype(jnp.float32)
        lc = _lane_cumsum(eq)
        rowsum = lc[:, C - 1:C]
        rp = _row_cumsum(rowsum)
        cume = fs[4] + (rp - rowsum) + (lc - eq)
        rstar = (is_[1] - is_[2]).astype(jnp.float32)
        sel = (eq > 0.0) & (cume == rstar)
        is_[3] = is_[3] + jnp.sum(jnp.where(sel, j, 0))
        fs[4] = fs[4] + jnp.sum(eq)

    @pl.when(s == 4 * NB - 1)
    def _emit():
        out_ref[0] = is_[3]


def kernel(logits):
    x = logits.reshape(1, N)
    xp = jnp.pad(x, ((0, 0), (0, NP - N)), constant_values=_NEG).reshape(R, C)

    p2d = pl.pallas_call(
        _softmax_kernel,
        grid=(3 * NB,),
        in_specs=[pl.BlockSpec((BR, C), lambda s: (s % NB, 0))],
        out_specs=pl.BlockSpec(
            (BR, C), lambda s: (jnp.where(s // NB < 2, 0, s % NB), 0)),
        out_shape=jax.ShapeDtypeStruct((R, C), jnp.float32),
        scratch_shapes=[pltpu.SMEM((2,), jnp.float32)],
    )(xp)

    ps = (-lax.sort(-p2d.reshape(1, NP), dimension=1)).reshape(R, C)
    g = jax.random.gumbel(jax.random.key(42), (1, N), jnp.float32)
    gp = jnp.pad(g, ((0, 0), (0, NP - N)), constant_values=_NEG).reshape(R, C)

    tok = pl.pallas_call(
        _sample_kernel,
        grid=(4 * NB,),
        in_specs=[
            pl.BlockSpec((BR, C), lambda s: (s % NB, 0)),
            pl.BlockSpec((BR, C), lambda s: (s % NB, 0)),
            pl.BlockSpec((BR, C), lambda s: (s % NB, 0)),
        ],
        out_specs=pl.BlockSpec(memory_space=pltpu.SMEM),
        out_shape=jax.ShapeDtypeStruct((1,), jnp.int32),
        scratch_shapes=[
            pltpu.SMEM((8,), jnp.float32),
            pltpu.SMEM((8,), jnp.int32),
        ],
    )(ps, gp, p2d)

    return tok


# speedup vs baseline: 6.8992x; 6.0318x over previous
import numpy as np
import jax
import jax.numpy as jnp
from jax import lax
from jax.experimental import pallas as pl
from jax.experimental.pallas import tpu as pltpu

N = 1_000_000
NP = 1 << 20
C = 128
R = NP // C
BR = 1024
NB = R // BR
BSZ = BR * C

_THRESH = np.float32(1.0 - 1.0 / 3.0)
_EPS = np.float32(1e-10)
_NEG = np.float32(-1e30)


def _lane_cumsum(blk):
    a = lax.broadcasted_iota(jnp.int32, (C, C), 0)
    b = lax.broadcasted_iota(jnp.int32, (C, C), 1)
    u = (a <= b).astype(jnp.float32)
    return lax.dot_general(blk, u, (((1,), (0,)), ((), ())),
                           preferred_element_type=jnp.float32)


def _row_cumsum(v):
    k = 1
    while k < BR:
        z = jnp.zeros((k, 1), v.dtype)
        v = v + jnp.concatenate([z, v[:-k]], axis=0)
        k *= 2
    return v


def _softmax_kernel(x_ref, p_ref, fs):
    s = pl.program_id(0)
    phase = s // NB

    @pl.when(s == 0)
    def _init():
        fs[0] = _NEG
        fs[1] = 0.0

    blk = x_ref[...]

    @pl.when(phase == 0)
    def _pmax():
        fs[0] = jnp.maximum(fs[0], jnp.max(blk))

    @pl.when(phase == 1)
    def _psum():
        fs[1] = fs[1] + jnp.sum(jnp.exp(blk - fs[0]))

    @pl.when(phase == 2)
    def _pwrite():
        p_ref[...] = jnp.exp(blk - fs[0]) / fs[1]


def _sample_kernel(ps_ref, g_ref, p_ref, out_ref, fs, is_):
    s = pl.program_id(0)
    phase = s // NB
    blk_id = s % NB

    @pl.when(s == 0)
    def _init():
        fs[0] = 0.0
        fs[1] = -1.0
        fs[2] = _NEG
        fs[3] = 0.0
        fs[4] = 0.0
        is_[0] = 0
        is_[1] = 0
        is_[2] = 0
        is_[3] = 0

    rr = lax.broadcasted_iota(jnp.int32, (BR, C), 0)
    cc = lax.broadcasted_iota(jnp.int32, (BR, C), 1)
    j = blk_id * BSZ + rr * C + cc

    @pl.when(phase == 0)
    def _p_cummass():
        ps = ps_ref[...]
        lc = _lane_cumsum(ps)
        rowsum = lc[:, C - 1:C]
        rp = _row_cumsum(rowsum)
        cum = fs[0] + (rp - rowsum) + lc
        kept = cum <= _THRESH
        is_[0] = is_[0] + jnp.sum(kept.astype(jnp.int32))
        fs[1] = jnp.maximum(fs[1], jnp.max(jnp.where(kept, cum, -1.0)))
        fs[0] = fs[0] + jnp.sum(rowsum)

    @pl.when(phase == 1)
    def _p_argmax():
        ps = ps_ref[...]
        g = g_ref[...]
        total = jnp.maximum(fs[1], _EPS)
        t = jnp.where(j < is_[0], ps / total, 0.0)
        sc = jnp.log(t + _EPS) + g
        bm = jnp.max(sc)
        bj = jnp.min(jnp.where(sc == bm, j, jnp.int32(2147483647)))

        @pl.when(bm > fs[2])
        def _upd():
            fs[2] = bm
            is_[1] = bj
            fs[3] = jnp.sum(jnp.where(j == bj, ps_ref[...], 0.0))

    @pl.when(phase == 2)
    def _p_gtcount():
        p = p_ref[...]
        is_[2] = is_[2] + jnp.sum((p > fs[3]).astype(jnp.int32))

    @pl.when(phase == 3)
    def _p_token():
        p = p_ref[...]
        eq = (p == fs[3]).astype(jnp.float32)
        lc = _lane_cumsum(eq)
        rowsum = lc[:, C - 1:C]
        rp = _row_cumsum(rowsum)
        cume = fs[4] + (rp - rowsum) + (lc - eq)
        rstar = (is_[1] - is_[2]).astype(jnp.float32)
        sel = (eq > 0.0) & (cume == rstar)
        is_[3] = is_[3] + jnp.sum(jnp.where(sel, j, 0))
        fs[4] = fs[4] + jnp.sum(eq)

    @pl.when(s == 4 * NB - 1)
    def _emit():
        out_ref[0] = is_[3]


def kernel(logits):
    x = logits.reshape(1, N)
    xp = jnp.pad(x, ((0, 0), (0, NP - N)), constant_values=_NEG).reshape(R, C)

    p2d = pl.pallas_call(
        _softmax_kernel,
        grid=(3 * NB,),
        in_specs=[pl.BlockSpec((BR, C), lambda s: (s % NB, 0))],
        out_specs=pl.BlockSpec(
            (BR, C), lambda s: (jnp.where(s // NB < 2, 0, s % NB), 0)),
        out_shape=jax.ShapeDtypeStruct((R, C), jnp.float32),
        scratch_shapes=[pltpu.SMEM((2,), jnp.float32)],
    )(xp)

    from jax.experimental.compute_on import compute_on

    @compute_on("tpu_sparsecore")
    @jax.jit
    def _scsort(v):
        return lax.sort(v, dimension=0)

    ps = (-_scsort(-p2d.reshape(NP))).reshape(R, C)
    g = jax.random.gumbel(jax.random.key(42), (1, N), jnp.float32)
    gp = jnp.pad(g, ((0, 0), (0, NP - N)), constant_values=_NEG).reshape(R, C)

    tok = pl.pallas_call(
        _sample_kernel,
        grid=(4 * NB,),
        in_specs=[
            pl.BlockSpec((BR, C), lambda s: (s % NB, 0)),
            pl.BlockSpec((BR, C), lambda s: (s % NB, 0)),
            pl.BlockSpec((BR, C), lambda s: (s % NB, 0)),
        ],
        out_specs=pl.BlockSpec(memory_space=pltpu.SMEM),
        out_shape=jax.ShapeDtypeStruct((1,), jnp.int32),
        scratch_shapes=[
            pltpu.SMEM((8,), jnp.float32),
            pltpu.SMEM((8,), jnp.int32),
        ],
    )(ps, gp, p2d)

    return tok
